# Initial kernel scaffold; baseline (speedup 1.0000x reference)
import jax
import jax.numpy as jnp
from jax.experimental import pallas as pl


def kernel(pred_s1, pred_s2, pred_s3, boxes, labels):
    raise NotImplementedError("write your pallas kernel here")


# speedup vs baseline: 3.1064x; 3.1064x over previous
import functools

import jax
import jax.numpy as jnp
from jax import lax
from jax.experimental import pallas as pl
from jax.experimental.pallas import tpu as pltpu
from jax.experimental.pallas import tpu_sc as plsc

_IMG_SIZE = 416.0
_NCLS = 80
_EPS = 1e-7
_B = 32
_NB = 20
_GRIDS = (13, 26, 52)
_ANCHORS = [[[116.0, 90.0], [156.0, 198.0], [373.0, 326.0]],
            [[30.0, 61.0], [62.0, 45.0], [59.0, 119.0]],
            [[10.0, 13.0], [16.0, 30.0], [33.0, 23.0]]]
_AW = [[a[0] / (_IMG_SIZE / g) for a in _ANCHORS[s]] for s, g in enumerate(_GRIDS)]
_AH = [[a[1] / (_IMG_SIZE / g) for a in _ANCHORS[s]] for s, g in enumerate(_GRIDS)]
_NCELLS = tuple(_B * 3 * g * g for g in _GRIDS)
_BLK = 1352


def _best_anchor(wg, hg, s):
    iou = []
    for a in range(3):
        inter = jnp.minimum(wg, _AW[s][a]) * jnp.minimum(hg, _AH[s][a])
        union = _AW[s][a] * _AH[s][a] + wg + hg - inter
        iou.append(jnp.where(union > 0, inter / union, 0.0))
    best = jnp.where(iou[1] > iou[0], jnp.full(wg.shape, 1, jnp.int32),
                     jnp.full(wg.shape, 0, jnp.int32))
    best = jnp.where(iou[2] > jnp.maximum(iou[0], iou[1]),
                     jnp.full(wg.shape, 2, jnp.int32), best)
    return best


def _dense_body(pref, oref):
    p = pref[:, 4:5]
    pc = jnp.clip(p, _EPS, 1.0 - _EPS)
    oref[...] = jnp.full((1, 1, 1), jnp.sum(-jnp.log(1.0 - pc)), jnp.float32)


def _dense_sum(pflat):
    nblk = pflat.shape[0] // _BLK
    return pl.pallas_call(
        _dense_body,
        grid=(nblk,),
        in_specs=[pl.BlockSpec((_BLK, 85), lambda i: (i, 0))],
        out_specs=pl.BlockSpec((1, 1, 1), lambda i: (i, 0, 0)),
        out_shape=jax.ShapeDtypeStruct((nblk, 1, 1), jnp.float32),
        compiler_params=pltpu.CompilerParams(
            dimension_semantics=("parallel",)),
    )(pflat)


def _sc_gather(boxes_t, p1f, p2f, p3f):
    mesh = plsc.VectorSubcoreMesh(core_axis_name="c", subcore_axis_name="s")

    @functools.partial(
        pl.kernel,
        mesh=mesh,
        out_type=(jax.ShapeDtypeStruct((3, _B, 32, 85), jnp.float32),
                  jax.ShapeDtypeStruct((3, _B, 32), jnp.int32)),
        scratch_types=[pltpu.VMEM((4, 32), jnp.float32),
                       pltpu.VMEM((3, 32), jnp.int32),
                       pltpu.VMEM((3, 32, 85), jnp.float32),
                       pltpu.SemaphoreType.DMA],
    )
    def body(boxes_hbm, p1, p2, p3, rows_out, idx_out, bx_v, idx_v, rows_v, sem):
        b = lax.axis_index("s") * 2 + lax.axis_index("c")
        pltpu.sync_copy(boxes_hbm.at[b], bx_v)
        lane = lax.iota(jnp.int32, 16)
        zero16 = jnp.full((16,), 0, jnp.int32)
        tabs = (p1, p2, p3)
        copies = []
        for s in range(3):
            g = _GRIDS[s]
            gf = jnp.float32(g)
            for k in range(2):
                xs = bx_v[0, pl.ds(k * 16, 16)]
                ys = bx_v[1, pl.ds(k * 16, 16)]
                ws = bx_v[2, pl.ds(k * 16, 16)]
                hs = bx_v[3, pl.ds(k * 16, 16)]
                fx = xs * gf
                fy = ys * gf
                gx = fx.astype(jnp.int32)
                gy = fy.astype(jnp.int32)
                gxc = jnp.minimum(gx, g - 1)
                gyc = jnp.minimum(gy, g - 1)
                best = _best_anchor(ws * gf, hs * gf, s)
                cell = ((b * 3 + best) * g + gyc) * g + gxc
                idx_v[s, pl.ds(k * 16, 16)] = cell
                for j in range(16 if k == 0 else _NB - 16):
                    cj = cell[j]
                    copies.append(pltpu.async_copy(
                        tabs[s].at[pl.ds(cj, 1)],
                        rows_v.at[s, pl.ds(k * 16 + j, 1)], sem))
        for cp in copies:
            cp.wait()
        for s in range(3):
            pltpu.sync_copy(rows_v.at[s], rows_out.at[s, b])
            pltpu.sync_copy(idx_v.at[s], idx_out.at[s, b])

    return body(boxes_t, p1f, p2f, p3f)


def _asm_body(parts1, parts2, parts3, boxes_ref, labels_ref, rows_ref, idx_ref,
              o_total, o_coord, o_obj, o_noobj, o_class):
    coord_loss = jnp.float32(0.0)
    obj_loss = jnp.float32(0.0)
    noobj_loss = jnp.float32(0.0)
    class_loss = jnp.float32(0.0)
    dense = (jnp.sum(parts1[...]), jnp.sum(parts2[...]), jnp.sum(parts3[...]))
    labels = labels_ref[...]
    for s in range(3):
        g = _GRIDS[s]
        gf = jnp.float32(g)
        x = boxes_ref[:, :, 0]
        y = boxes_ref[:, :, 1]
        w = boxes_ref[:, :, 2]
        h = boxes_ref[:, :, 3]
        fx = x * gf
        fy = y * gf
        gx = fx.astype(jnp.int32)
        gy = fy.astype(jnp.int32)
        valid = (gx < g) & (gy < g)
        tx = fx - gx.astype(jnp.float32)
        ty = fy - gy.astype(jnp.float32)
        wg = w * gf
        hg = h * gf
        best = _best_anchor(wg, hg, s)
        awb = jnp.where(best == 1, _AW[s][1], _AW[s][0])
        awb = jnp.where(best == 2, _AW[s][2], awb)
        ahb = jnp.where(best == 1, _AH[s][1], _AH[s][0])
        ahb = jnp.where(best == 2, _AH[s][2], ahb)
        tw = wg / awb
        th = hg / ahb
        key = idx_ref[s][:, :_NB]
        eq = key[:, :, None] == key[:, None, :]
        ii = lax.broadcasted_iota(jnp.int32, (_B, _NB, _NB), 1)
        jj = lax.broadcasted_iota(jnp.int32, (_B, _NB, _NB), 2)
        conflict = jnp.any(eq & (jj > ii) & valid[:, None, :], axis=-1)
        winner = valid & ~conflict
        wm = winner.astype(jnp.float32)
        n_obj = jnp.sum(wm)
        rows = rows_ref[s][:, :_NB, :]
        px = rows[:, :, 0]
        py = rows[:, :, 1]
        pw = rows[:, :, 2]
        ph = rows[:, :, 3]
        pobj = rows[:, :, 4]
        pcls = rows[:, :, 5:]
        n_div = jnp.maximum(n_obj * 2.0, 1.0)
        mse_xy = jnp.sum(wm * ((px - tx) ** 2 + (py - ty) ** 2)) / n_div
        mse_wh = jnp.sum(wm * ((jnp.sqrt(pw) - jnp.sqrt(tw)) ** 2
                               + (jnp.sqrt(ph) - jnp.sqrt(th)) ** 2)) / n_div
        has_obj = (n_obj > 0).astype(jnp.float32)
        coord_loss = coord_loss + has_obj * (mse_xy + mse_wh)
        pobj_c = jnp.clip(pobj, _EPS, 1.0 - _EPS)
        obj_loss = obj_loss + jnp.sum(wm * (-jnp.log(pobj_c))) / jnp.maximum(n_obj, 1.0)
        corr = jnp.sum(wm * (-jnp.log(1.0 - pobj_c)))
        n_noobj = _NCELLS[s] - n_obj
        noobj_loss = noobj_loss + (dense[s] - corr) / jnp.maximum(n_noobj, 1.0)
        cidx = lax.broadcasted_iota(jnp.int32, (_B, _NB, _NCLS), 2)
        onehot = (cidx == labels[:, :, None]).astype(jnp.float32)
        pc = jnp.clip(pcls, _EPS, 1.0 - _EPS)
        bce = -(onehot * jnp.log(pc) + (1.0 - onehot) * jnp.log(1.0 - pc))
        class_loss = class_loss + has_obj * (
            jnp.sum(wm[:, :, None] * bce) / jnp.maximum(n_obj * _NCLS, 1.0))
    total = (5.0 * coord_loss + obj_loss + 0.5 * noobj_loss + class_loss) / _B
    o_total[...] = jnp.full((1, 1), total, jnp.float32)
    o_coord[...] = jnp.full((1, 1), coord_loss / _B, jnp.float32)
    o_obj[...] = jnp.full((1, 1), obj_loss / _B, jnp.float32)
    o_noobj[...] = jnp.full((1, 1), noobj_loss / _B, jnp.float32)
    o_class[...] = jnp.full((1, 1), class_loss / _B, jnp.float32)


def _assembly(parts1, parts2, parts3, boxes, labels, rows, cellidx):
    sd = jax.ShapeDtypeStruct((1, 1), jnp.float32)
    return pl.pallas_call(
        _asm_body,
        out_shape=(sd, sd, sd, sd, sd),
    )(parts1, parts2, parts3, boxes, labels, rows, cellidx)


def kernel(pred_s1, pred_s2, pred_s3, boxes, labels):
    p1f = pred_s1.reshape(-1, 85)
    p2f = pred_s2.reshape(-1, 85)
    p3f = pred_s3.reshape(-1, 85)
    parts1 = _dense_sum(p1f)
    parts2 = _dense_sum(p2f)
    parts3 = _dense_sum(p3f)
    boxes_t = jnp.pad(boxes, ((0, 0), (0, 32 - _NB), (0, 0)),
                      mode="edge").transpose(0, 2, 1)
    rows, cellidx = _sc_gather(boxes_t, p1f, p2f, p3f)
    t, c, o, n, cl = _assembly(parts1, parts2, parts3, boxes,
                               labels.astype(jnp.int32), rows, cellidx)
    return (t.reshape(()), c.reshape(()), o.reshape(()),
            n.reshape(()), cl.reshape(()))


# speedup vs baseline: 6.6359x; 2.1362x over previous
import functools

import jax
import jax.numpy as jnp
from jax import lax
from jax.experimental import pallas as pl
from jax.experimental.pallas import tpu as pltpu
from jax.experimental.pallas import tpu_sc as plsc

_IMG_SIZE = 416.0
_NCLS = 80
_EPS = 1e-7
_B = 32
_NB = 20
_GRIDS = (13, 26, 52)
_ANCHORS = [[[116.0, 90.0], [156.0, 198.0], [373.0, 326.0]],
            [[30.0, 61.0], [62.0, 45.0], [59.0, 119.0]],
            [[10.0, 13.0], [16.0, 30.0], [33.0, 23.0]]]
_AW = [[a[0] / (_IMG_SIZE / g) for a in _ANCHORS[s]] for s, g in enumerate(_GRIDS)]
_AH = [[a[1] / (_IMG_SIZE / g) for a in _ANCHORS[s]] for s, g in enumerate(_GRIDS)]
_NCELLS = tuple(_B * 3 * g * g for g in _GRIDS)
_BLK = 1352


def _best_anchor(wg, hg, s):
    iou = []
    for a in range(3):
        inter = jnp.minimum(wg, _AW[s][a]) * jnp.minimum(hg, _AH[s][a])
        union = _AW[s][a] * _AH[s][a] + wg + hg - inter
        iou.append(jnp.where(union > 0, inter / union, 0.0))
    best = jnp.where(iou[1] > iou[0], jnp.full(wg.shape, 1, jnp.int32),
                     jnp.full(wg.shape, 0, jnp.int32))
    best = jnp.where(iou[2] > jnp.maximum(iou[0], iou[1]),
                     jnp.full(wg.shape, 2, jnp.int32), best)
    return best


def _dense_body(pref, oref):
    p = pref[:, :, :, :, 4:5]
    pc = jnp.clip(p, _EPS, 1.0 - _EPS)
    oref[...] = jnp.full((1, 1, 1), jnp.sum(-jnp.log(1.0 - pc)), jnp.float32)


def _dense_sum(pred, blk_b):
    g = pred.shape[2]
    nblk = _B // blk_b
    return pl.pallas_call(
        _dense_body,
        grid=(nblk,),
        in_specs=[pl.BlockSpec((blk_b, 3, g, g, 85),
                               lambda i: (i, 0, 0, 0, 0))],
        out_specs=pl.BlockSpec((1, 1, 1), lambda i: (i, 0, 0)),
        out_shape=jax.ShapeDtypeStruct((nblk, 1, 1), jnp.float32),
        compiler_params=pltpu.CompilerParams(
            dimension_semantics=("parallel",)),
    )(pred)


def _sc_gather(boxes_t, p1f, p2f, p3f):
    mesh = plsc.VectorSubcoreMesh(core_axis_name="c", subcore_axis_name="s")

    @functools.partial(
        pl.kernel,
        mesh=mesh,
        out_type=(jax.ShapeDtypeStruct((3, _B, 32, 85), jnp.float32),
                  jax.ShapeDtypeStruct((3, _B, 32), jnp.int32)),
        scratch_types=[pltpu.VMEM((4, 32), jnp.float32),
                       pltpu.VMEM((3, 32), jnp.int32),
                       pltpu.VMEM((3, 32, 85), jnp.float32),
                       pltpu.SemaphoreType.DMA],
    )
    def body(boxes_hbm, p1, p2, p3, rows_out, idx_out, bx_v, idx_v, rows_v, sem):
        b = lax.axis_index("s") * 2 + lax.axis_index("c")
        pltpu.sync_copy(boxes_hbm.at[b], bx_v)
        lane = lax.iota(jnp.int32, 16)
        zero16 = jnp.full((16,), 0, jnp.int32)
        tabs = (p1, p2, p3)
        copies = []
        for s in range(3):
            g = _GRIDS[s]
            gf = jnp.float32(g)
            for k in range(2):
                xs = bx_v[0, pl.ds(k * 16, 16)]
                ys = bx_v[1, pl.ds(k * 16, 16)]
                ws = bx_v[2, pl.ds(k * 16, 16)]
                hs = bx_v[3, pl.ds(k * 16, 16)]
                fx = xs * gf
                fy = ys * gf
                gx = fx.astype(jnp.int32)
                gy = fy.astype(jnp.int32)
                gxc = jnp.minimum(gx, g - 1)
                gyc = jnp.minimum(gy, g - 1)
                best = _best_anchor(ws * gf, hs * gf, s)
                cell = ((b * 3 + best) * g + gyc) * g + gxc
                idx_v[s, pl.ds(k * 16, 16)] = cell
                for j in range(16 if k == 0 else _NB - 16):
                    copies.append(pltpu.async_copy(
                        tabs[s].at[b, best[j], gyc[j], gxc[j]],
                        rows_v.at[s, k * 16 + j], sem))
        for cp in copies:
            cp.wait()
        for s in range(3):
            pltpu.sync_copy(rows_v.at[s], rows_out.at[s, b])
            pltpu.sync_copy(idx_v.at[s], idx_out.at[s, b])

    return body(boxes_t, p1f, p2f, p3f)


def _asm_body(parts1, parts2, parts3, boxes_ref, labels_ref, rows_ref, idx_ref,
              o_total, o_coord, o_obj, o_noobj, o_class):
    coord_loss = jnp.float32(0.0)
    obj_loss = jnp.float32(0.0)
    noobj_loss = jnp.float32(0.0)
    class_loss = jnp.float32(0.0)
    dense = (jnp.sum(parts1[...]), jnp.sum(parts2[...]), jnp.sum(parts3[...]))
    labels = labels_ref[...]
    for s in range(3):
        g = _GRIDS[s]
        gf = jnp.float32(g)
        x = boxes_ref[:, :, 0]
        y = boxes_ref[:, :, 1]
        w = boxes_ref[:, :, 2]
        h = boxes_ref[:, :, 3]
        fx = x * gf
        fy = y * gf
        gx = fx.astype(jnp.int32)
        gy = fy.astype(jnp.int32)
        valid = (gx < g) & (gy < g)
        tx = fx - gx.astype(jnp.float32)
        ty = fy - gy.astype(jnp.float32)
        wg = w * gf
        hg = h * gf
        best = _best_anchor(wg, hg, s)
        awb = jnp.where(best == 1, _AW[s][1], _AW[s][0])
        awb = jnp.where(best == 2, _AW[s][2], awb)
        ahb = jnp.where(best == 1, _AH[s][1], _AH[s][0])
        ahb = jnp.where(best == 2, _AH[s][2], ahb)
        tw = wg / awb
        th = hg / ahb
        key = idx_ref[s][:, :_NB]
        eq = key[:, :, None] == key[:, None, :]
        ii = lax.broadcasted_iota(jnp.int32, (_B, _NB, _NB), 1)
        jj = lax.broadcasted_iota(jnp.int32, (_B, _NB, _NB), 2)
        conflict = jnp.any(eq & (jj > ii) & valid[:, None, :], axis=-1)
        winner = valid & ~conflict
        wm = winner.astype(jnp.float32)
        n_obj = jnp.sum(wm)
        rows = rows_ref[s][:, :_NB, :]
        px = rows[:, :, 0]
        py = rows[:, :, 1]
        pw = rows[:, :, 2]
        ph = rows[:, :, 3]
        pobj = rows[:, :, 4]
        pcls = rows[:, :, 5:]
        n_div = jnp.maximum(n_obj * 2.0, 1.0)
        mse_xy = jnp.sum(wm * ((px - tx) ** 2 + (py - ty) ** 2)) / n_div
        mse_wh = jnp.sum(wm * ((jnp.sqrt(pw) - jnp.sqrt(tw)) ** 2
                               + (jnp.sqrt(ph) - jnp.sqrt(th)) ** 2)) / n_div
        has_obj = (n_obj > 0).astype(jnp.float32)
        coord_loss = coord_loss + has_obj * (mse_xy + mse_wh)
        pobj_c = jnp.clip(pobj, _EPS, 1.0 - _EPS)
        obj_loss = obj_loss + jnp.sum(wm * (-jnp.log(pobj_c))) / jnp.maximum(n_obj, 1.0)
        corr = jnp.sum(wm * (-jnp.log(1.0 - pobj_c)))
        n_noobj = _NCELLS[s] - n_obj
        noobj_loss = noobj_loss + (dense[s] - corr) / jnp.maximum(n_noobj, 1.0)
        cidx = lax.broadcasted_iota(jnp.int32, (_B, _NB, _NCLS), 2)
        onehot = (cidx == labels[:, :, None]).astype(jnp.float32)
        pc = jnp.clip(pcls, _EPS, 1.0 - _EPS)
        bce = -(onehot * jnp.log(pc) + (1.0 - onehot) * jnp.log(1.0 - pc))
        class_loss = class_loss + has_obj * (
            jnp.sum(wm[:, :, None] * bce) / jnp.maximum(n_obj * _NCLS, 1.0))
    total = (5.0 * coord_loss + obj_loss + 0.5 * noobj_loss + class_loss) / _B
    o_total[...] = jnp.full((1, 1), total, jnp.float32)
    o_coord[...] = jnp.full((1, 1), coord_loss / _B, jnp.float32)
    o_obj[...] = jnp.full((1, 1), obj_loss / _B, jnp.float32)
    o_noobj[...] = jnp.full((1, 1), noobj_loss / _B, jnp.float32)
    o_class[...] = jnp.full((1, 1), class_loss / _B, jnp.float32)


def _assembly(parts1, parts2, parts3, boxes, labels, rows, cellidx):
    sd = jax.ShapeDtypeStruct((1, 1), jnp.float32)
    return pl.pallas_call(
        _asm_body,
        out_shape=(sd, sd, sd, sd, sd),
    )(parts1, parts2, parts3, boxes, labels, rows, cellidx)


def kernel(pred_s1, pred_s2, pred_s3, boxes, labels):
    parts1 = _dense_sum(pred_s1, 8)
    parts2 = _dense_sum(pred_s2, 2)
    parts3 = _dense_sum(pred_s3, 1)
    boxes_t = jnp.pad(boxes, ((0, 0), (0, 32 - _NB), (0, 0)),
                      mode="edge").transpose(0, 2, 1)
    rows, cellidx = _sc_gather(boxes_t, pred_s1, pred_s2, pred_s3)
    t, c, o, n, cl = _assembly(parts1, parts2, parts3, boxes,
                               labels.astype(jnp.int32), rows, cellidx)
    return (t.reshape(()), c.reshape(()), o.reshape(()),
            n.reshape(()), cl.reshape(()))


# speedup vs baseline: 7.0646x; 1.0646x over previous
import functools

import jax
import jax.numpy as jnp
from jax import lax
from jax.experimental import pallas as pl
from jax.experimental.pallas import tpu as pltpu
from jax.experimental.pallas import tpu_sc as plsc

_IMG_SIZE = 416.0
_NCLS = 80
_EPS = 1e-7
_B = 32
_NB = 20
_GRIDS = (13, 26, 52)
_ANCHORS = [[[116.0, 90.0], [156.0, 198.0], [373.0, 326.0]],
            [[30.0, 61.0], [62.0, 45.0], [59.0, 119.0]],
            [[10.0, 13.0], [16.0, 30.0], [33.0, 23.0]]]
_AW = [[a[0] / (_IMG_SIZE / g) for a in _ANCHORS[s]] for s, g in enumerate(_GRIDS)]
_AH = [[a[1] / (_IMG_SIZE / g) for a in _ANCHORS[s]] for s, g in enumerate(_GRIDS)]
_NCELLS = tuple(_B * 3 * g * g for g in _GRIDS)
_BLK = 1352


def _best_anchor(wg, hg, s):
    iou = []
    for a in range(3):
        inter = jnp.minimum(wg, _AW[s][a]) * jnp.minimum(hg, _AH[s][a])
        union = _AW[s][a] * _AH[s][a] + wg + hg - inter
        iou.append(jnp.where(union > 0, inter / union, 0.0))
    best = jnp.where(iou[1] > iou[0], jnp.full(wg.shape, 1, jnp.int32),
                     jnp.full(wg.shape, 0, jnp.int32))
    best = jnp.where(iou[2] > jnp.maximum(iou[0], iou[1]),
                     jnp.full(wg.shape, 2, jnp.int32), best)
    return best


def _dense_body(p1ref, p2ref, p3ref, o1ref, o2ref, o3ref):
    for pref, oref in ((p1ref, o1ref), (p2ref, o2ref), (p3ref, o3ref)):
        p = pref[:, :, :, :, 4:5]
        pc = jnp.clip(p, _EPS, 1.0 - _EPS)
        oref[...] = jnp.full((1, 1, 1), jnp.sum(-jnp.log(1.0 - pc)),
                             jnp.float32)


def _dense_sum(pred_s1, pred_s2, pred_s3):
    sd = jax.ShapeDtypeStruct((_B, 1, 1), jnp.float32)
    return pl.pallas_call(
        _dense_body,
        grid=(_B,),
        in_specs=[pl.BlockSpec((1, 3, g, g, 85), lambda i: (i, 0, 0, 0, 0))
                  for g in _GRIDS],
        out_specs=[pl.BlockSpec((1, 1, 1), lambda i: (i, 0, 0))] * 3,
        out_shape=(sd, sd, sd),
        compiler_params=pltpu.CompilerParams(
            dimension_semantics=("parallel",)),
    )(pred_s1, pred_s2, pred_s3)


def _sc_gather(boxes_t, p1f, p2f, p3f):
    mesh = plsc.VectorSubcoreMesh(core_axis_name="c", subcore_axis_name="s")

    @functools.partial(
        pl.kernel,
        mesh=mesh,
        out_type=(jax.ShapeDtypeStruct((3, _B, 32, 85), jnp.float32),
                  jax.ShapeDtypeStruct((3, _B, 32), jnp.int32)),
        scratch_types=[pltpu.VMEM((4, 32), jnp.float32),
                       pltpu.VMEM((3, 32), jnp.int32),
                       pltpu.VMEM((3, 32, 85), jnp.float32),
                       pltpu.SemaphoreType.DMA],
    )
    def body(boxes_hbm, p1, p2, p3, rows_out, idx_out, bx_v, idx_v, rows_v, sem):
        b = lax.axis_index("s") * 2 + lax.axis_index("c")
        pltpu.sync_copy(boxes_hbm.at[b], bx_v)
        lane = lax.iota(jnp.int32, 16)
        zero16 = jnp.full((16,), 0, jnp.int32)
        tabs = (p1, p2, p3)
        copies = []
        for s in range(3):
            g = _GRIDS[s]
            gf = jnp.float32(g)
            for k in range(2):
                xs = bx_v[0, pl.ds(k * 16, 16)]
                ys = bx_v[1, pl.ds(k * 16, 16)]
                ws = bx_v[2, pl.ds(k * 16, 16)]
                hs = bx_v[3, pl.ds(k * 16, 16)]
                fx = xs * gf
                fy = ys * gf
                gx = fx.astype(jnp.int32)
                gy = fy.astype(jnp.int32)
                gxc = jnp.minimum(gx, g - 1)
                gyc = jnp.minimum(gy, g - 1)
                best = _best_anchor(ws * gf, hs * gf, s)
                cell = ((b * 3 + best) * g + gyc) * g + gxc
                idx_v[s, pl.ds(k * 16, 16)] = cell
                for j in range(16 if k == 0 else _NB - 16):
                    copies.append(pltpu.async_copy(
                        tabs[s].at[b, best[j], gyc[j], gxc[j]],
                        rows_v.at[s, k * 16 + j], sem))
        for cp in copies:
            cp.wait()
        for s in range(3):
            pltpu.sync_copy(rows_v.at[s], rows_out.at[s, b])
            pltpu.sync_copy(idx_v.at[s], idx_out.at[s, b])

    return body(boxes_t, p1f, p2f, p3f)


def _asm_body(parts1, parts2, parts3, boxes_ref, labels_ref, rows_ref, idx_ref,
              o_total, o_coord, o_obj, o_noobj, o_class):
    coord_loss = jnp.float32(0.0)
    obj_loss = jnp.float32(0.0)
    noobj_loss = jnp.float32(0.0)
    class_loss = jnp.float32(0.0)
    dense = (jnp.sum(parts1[...]), jnp.sum(parts2[...]), jnp.sum(parts3[...]))
    labels = labels_ref[...]
    for s in range(3):
        g = _GRIDS[s]
        gf = jnp.float32(g)
        x = boxes_ref[:, :, 0]
        y = boxes_ref[:, :, 1]
        w = boxes_ref[:, :, 2]
        h = boxes_ref[:, :, 3]
        fx = x * gf
        fy = y * gf
        gx = fx.astype(jnp.int32)
        gy = fy.astype(jnp.int32)
        valid = (gx < g) & (gy < g)
        tx = fx - gx.astype(jnp.float32)
        ty = fy - gy.astype(jnp.float32)
        wg = w * gf
        hg = h * gf
        best = _best_anchor(wg, hg, s)
        awb = jnp.where(best == 1, _AW[s][1], _AW[s][0])
        awb = jnp.where(best == 2, _AW[s][2], awb)
        ahb = jnp.where(best == 1, _AH[s][1], _AH[s][0])
        ahb = jnp.where(best == 2, _AH[s][2], ahb)
        tw = wg / awb
        th = hg / ahb
        key = idx_ref[s][:, :_NB]
        eq = key[:, :, None] == key[:, None, :]
        ii = lax.broadcasted_iota(jnp.int32, (_B, _NB, _NB), 1)
        jj = lax.broadcasted_iota(jnp.int32, (_B, _NB, _NB), 2)
        conflict = jnp.any(eq & (jj > ii) & valid[:, None, :], axis=-1)
        winner = valid & ~conflict
        wm = winner.astype(jnp.float32)
        n_obj = jnp.sum(wm)
        rows = rows_ref[s][:, :_NB, :]
        px = rows[:, :, 0]
        py = rows[:, :, 1]
        pw = rows[:, :, 2]
        ph = rows[:, :, 3]
        pobj = rows[:, :, 4]
        pcls = rows[:, :, 5:]
        n_div = jnp.maximum(n_obj * 2.0, 1.0)
        mse_xy = jnp.sum(wm * ((px - tx) ** 2 + (py - ty) ** 2)) / n_div
        mse_wh = jnp.sum(wm * ((jnp.sqrt(pw) - jnp.sqrt(tw)) ** 2
                               + (jnp.sqrt(ph) - jnp.sqrt(th)) ** 2)) / n_div
        has_obj = (n_obj > 0).astype(jnp.float32)
        coord_loss = coord_loss + has_obj * (mse_xy + mse_wh)
        pobj_c = jnp.clip(pobj, _EPS, 1.0 - _EPS)
        obj_loss = obj_loss + jnp.sum(wm * (-jnp.log(pobj_c))) / jnp.maximum(n_obj, 1.0)
        corr = jnp.sum(wm * (-jnp.log(1.0 - pobj_c)))
        n_noobj = _NCELLS[s] - n_obj
        noobj_loss = noobj_loss + (dense[s] - corr) / jnp.maximum(n_noobj, 1.0)
        cidx = lax.broadcasted_iota(jnp.int32, (_B, _NB, _NCLS), 2)
        onehot = (cidx == labels[:, :, None]).astype(jnp.float32)
        pc = jnp.clip(pcls, _EPS, 1.0 - _EPS)
        bce = -(onehot * jnp.log(pc) + (1.0 - onehot) * jnp.log(1.0 - pc))
        class_loss = class_loss + has_obj * (
            jnp.sum(wm[:, :, None] * bce) / jnp.maximum(n_obj * _NCLS, 1.0))
    total = (5.0 * coord_loss + obj_loss + 0.5 * noobj_loss + class_loss) / _B
    o_total[...] = jnp.full((1, 1), total, jnp.float32)
    o_coord[...] = jnp.full((1, 1), coord_loss / _B, jnp.float32)
    o_obj[...] = jnp.full((1, 1), obj_loss / _B, jnp.float32)
    o_noobj[...] = jnp.full((1, 1), noobj_loss / _B, jnp.float32)
    o_class[...] = jnp.full((1, 1), class_loss / _B, jnp.float32)


def _assembly(parts1, parts2, parts3, boxes, labels, rows, cellidx):
    sd = jax.ShapeDtypeStruct((1, 1), jnp.float32)
    return pl.pallas_call(
        _asm_body,
        out_shape=(sd, sd, sd, sd, sd),
    )(parts1, parts2, parts3, boxes, labels, rows, cellidx)


def kernel(pred_s1, pred_s2, pred_s3, boxes, labels):
    parts1, parts2, parts3 = _dense_sum(pred_s1, pred_s2, pred_s3)
    boxes_t = jnp.pad(boxes, ((0, 0), (0, 32 - _NB), (0, 0)),
                      mode="edge").transpose(0, 2, 1)
    rows, cellidx = _sc_gather(boxes_t, pred_s1, pred_s2, pred_s3)
    t, c, o, n, cl = _assembly(parts1, parts2, parts3, boxes,
                               labels.astype(jnp.int32), rows, cellidx)
    return (t.reshape(()), c.reshape(()), o.reshape(()),
            n.reshape(()), cl.reshape(()))


# speedup vs baseline: 7.0674x; 1.0004x over previous
import functools

import jax
import jax.numpy as jnp
from jax import lax
from jax.experimental import pallas as pl
from jax.experimental.pallas import tpu as pltpu
from jax.experimental.pallas import tpu_sc as plsc

_IMG_SIZE = 416.0
_NCLS = 80
_EPS = 1e-7
_B = 32
_NB = 20
_GRIDS = (13, 26, 52)
_ANCHORS = [[[116.0, 90.0], [156.0, 198.0], [373.0, 326.0]],
            [[30.0, 61.0], [62.0, 45.0], [59.0, 119.0]],
            [[10.0, 13.0], [16.0, 30.0], [33.0, 23.0]]]
_AW = [[a[0] / (_IMG_SIZE / g) for a in _ANCHORS[s]] for s, g in enumerate(_GRIDS)]
_AH = [[a[1] / (_IMG_SIZE / g) for a in _ANCHORS[s]] for s, g in enumerate(_GRIDS)]
_NCELLS = tuple(_B * 3 * g * g for g in _GRIDS)
_BLK = 1352


def _best_anchor(wg, hg, s):
    iou = []
    for a in range(3):
        inter = jnp.minimum(wg, _AW[s][a]) * jnp.minimum(hg, _AH[s][a])
        union = _AW[s][a] * _AH[s][a] + wg + hg - inter
        iou.append(jnp.where(union > 0, inter / union, 0.0))
    best = jnp.where(iou[1] > iou[0], jnp.full(wg.shape, 1, jnp.int32),
                     jnp.full(wg.shape, 0, jnp.int32))
    best = jnp.where(iou[2] > jnp.maximum(iou[0], iou[1]),
                     jnp.full(wg.shape, 2, jnp.int32), best)
    return best


def _dense_body(p1ref, p2ref, p3ref, o1ref, o2ref, o3ref):
    for pref, oref in ((p1ref, o1ref), (p2ref, o2ref), (p3ref, o3ref)):
        p = pref[:, :, :, :, 4:5]
        pc = jnp.clip(p, _EPS, 1.0 - _EPS)
        oref[...] = jnp.full((1, 1, 1), jnp.sum(-jnp.log(1.0 - pc)),
                             jnp.float32)


def _dense_sum(pred_s1, pred_s2, pred_s3):
    sd = jax.ShapeDtypeStruct((_B, 1, 1), jnp.float32)
    return pl.pallas_call(
        _dense_body,
        grid=(_B,),
        in_specs=[pl.BlockSpec((1, 3, g, g, 85), lambda i: (i, 0, 0, 0, 0))
                  for g in _GRIDS],
        out_specs=[pl.BlockSpec((1, 1, 1), lambda i: (i, 0, 0))] * 3,
        out_shape=(sd, sd, sd),
        compiler_params=pltpu.CompilerParams(
            dimension_semantics=("parallel",)),
    )(pred_s1, pred_s2, pred_s3)


def _sc_gather(boxes_t, p1f, p2f, p3f):
    mesh = plsc.VectorSubcoreMesh(core_axis_name="c", subcore_axis_name="s")

    @functools.partial(
        pl.kernel,
        mesh=mesh,
        out_type=(jax.ShapeDtypeStruct((3, _B, 32, 85), jnp.float32),
                  jax.ShapeDtypeStruct((3, _B, 32), jnp.int32)),
        scratch_types=[pltpu.VMEM((4, 32), jnp.float32),
                       pltpu.VMEM((3, 32), jnp.int32),
                       pltpu.VMEM((3, 32, 85), jnp.float32),
                       pltpu.SemaphoreType.DMA],
        compiler_params=pltpu.CompilerParams(use_tc_tiling_on_sc=True),
    )
    def body(boxes_hbm, p1, p2, p3, rows_out, idx_out, bx_v, idx_v, rows_v, sem):
        b = lax.axis_index("s") * 2 + lax.axis_index("c")
        pltpu.sync_copy(boxes_hbm.at[b], bx_v)
        lane = lax.iota(jnp.int32, 16)
        zero16 = jnp.full((16,), 0, jnp.int32)
        tabs = (p1, p2, p3)
        copies = []
        for s in range(3):
            g = _GRIDS[s]
            gf = jnp.float32(g)
            for k in range(2):
                xs = bx_v[0, pl.ds(k * 16, 16)]
                ys = bx_v[1, pl.ds(k * 16, 16)]
                ws = bx_v[2, pl.ds(k * 16, 16)]
                hs = bx_v[3, pl.ds(k * 16, 16)]
                fx = xs * gf
                fy = ys * gf
                gx = fx.astype(jnp.int32)
                gy = fy.astype(jnp.int32)
                gxc = jnp.minimum(gx, g - 1)
                gyc = jnp.minimum(gy, g - 1)
                best = _best_anchor(ws * gf, hs * gf, s)
                cell = ((b * 3 + best) * g + gyc) * g + gxc
                idx_v[s, pl.ds(k * 16, 16)] = cell
                for j in range(16 if k == 0 else _NB - 16):
                    copies.append(pltpu.async_copy(
                        tabs[s].at[b, best[j], gyc[j], gxc[j]],
                        rows_v.at[s, k * 16 + j], sem))
        for cp in copies:
            cp.wait()
        for s in range(3):
            pltpu.sync_copy(rows_v.at[s], rows_out.at[s, b])
            pltpu.sync_copy(idx_v.at[s], idx_out.at[s, b])

    return body(boxes_t, p1f, p2f, p3f)


def _asm_body(parts1, parts2, parts3, boxes_ref, labels_ref, rows_ref, idx_ref,
              o_total, o_coord, o_obj, o_noobj, o_class):
    coord_loss = jnp.float32(0.0)
    obj_loss = jnp.float32(0.0)
    noobj_loss = jnp.float32(0.0)
    class_loss = jnp.float32(0.0)
    dense = (jnp.sum(parts1[...]), jnp.sum(parts2[...]), jnp.sum(parts3[...]))
    labels = labels_ref[...]
    for s in range(3):
        g = _GRIDS[s]
        gf = jnp.float32(g)
        x = boxes_ref[:, :, 0]
        y = boxes_ref[:, :, 1]
        w = boxes_ref[:, :, 2]
        h = boxes_ref[:, :, 3]
        fx = x * gf
        fy = y * gf
        gx = fx.astype(jnp.int32)
        gy = fy.astype(jnp.int32)
        valid = (gx < g) & (gy < g)
        tx = fx - gx.astype(jnp.float32)
        ty = fy - gy.astype(jnp.float32)
        wg = w * gf
        hg = h * gf
        best = _best_anchor(wg, hg, s)
        awb = jnp.where(best == 1, _AW[s][1], _AW[s][0])
        awb = jnp.where(best == 2, _AW[s][2], awb)
        ahb = jnp.where(best == 1, _AH[s][1], _AH[s][0])
        ahb = jnp.where(best == 2, _AH[s][2], ahb)
        tw = wg / awb
        th = hg / ahb
        key = idx_ref[s][:, :_NB]
        eq = key[:, :, None] == key[:, None, :]
        ii = lax.broadcasted_iota(jnp.int32, (_B, _NB, _NB), 1)
        jj = lax.broadcasted_iota(jnp.int32, (_B, _NB, _NB), 2)
        conflict = jnp.any(eq & (jj > ii) & valid[:, None, :], axis=-1)
        winner = valid & ~conflict
        wm = winner.astype(jnp.float32)
        n_obj = jnp.sum(wm)
        rows = rows_ref[s][:, :_NB, :]
        px = rows[:, :, 0]
        py = rows[:, :, 1]
        pw = rows[:, :, 2]
        ph = rows[:, :, 3]
        pobj = rows[:, :, 4]
        pcls = rows[:, :, 5:]
        n_div = jnp.maximum(n_obj * 2.0, 1.0)
        mse_xy = jnp.sum(wm * ((px - tx) ** 2 + (py - ty) ** 2)) / n_div
        mse_wh = jnp.sum(wm * ((jnp.sqrt(pw) - jnp.sqrt(tw)) ** 2
                               + (jnp.sqrt(ph) - jnp.sqrt(th)) ** 2)) / n_div
        has_obj = (n_obj > 0).astype(jnp.float32)
        coord_loss = coord_loss + has_obj * (mse_xy + mse_wh)
        pobj_c = jnp.clip(pobj, _EPS, 1.0 - _EPS)
        obj_loss = obj_loss + jnp.sum(wm * (-jnp.log(pobj_c))) / jnp.maximum(n_obj, 1.0)
        corr = jnp.sum(wm * (-jnp.log(1.0 - pobj_c)))
        n_noobj = _NCELLS[s] - n_obj
        noobj_loss = noobj_loss + (dense[s] - corr) / jnp.maximum(n_noobj, 1.0)
        cidx = lax.broadcasted_iota(jnp.int32, (_B, _NB, _NCLS), 2)
        onehot = (cidx == labels[:, :, None]).astype(jnp.float32)
        pc = jnp.clip(pcls, _EPS, 1.0 - _EPS)
        bce = -(onehot * jnp.log(pc) + (1.0 - onehot) * jnp.log(1.0 - pc))
        class_loss = class_loss + has_obj * (
            jnp.sum(wm[:, :, None] * bce) / jnp.maximum(n_obj * _NCLS, 1.0))
    total = (5.0 * coord_loss + obj_loss + 0.5 * noobj_loss + class_loss) / _B
    o_total[...] = jnp.full((1, 1), total, jnp.float32)
    o_coord[...] = jnp.full((1, 1), coord_loss / _B, jnp.float32)
    o_obj[...] = jnp.full((1, 1), obj_loss / _B, jnp.float32)
    o_noobj[...] = jnp.full((1, 1), noobj_loss / _B, jnp.float32)
    o_class[...] = jnp.full((1, 1), class_loss / _B, jnp.float32)


def _assembly(parts1, parts2, parts3, boxes, labels, rows, cellidx):
    sd = jax.ShapeDtypeStruct((1, 1), jnp.float32)
    return pl.pallas_call(
        _asm_body,
        out_shape=(sd, sd, sd, sd, sd),
    )(parts1, parts2, parts3, boxes, labels, rows, cellidx)


def kernel(pred_s1, pred_s2, pred_s3, boxes, labels):
    parts1, parts2, parts3 = _dense_sum(pred_s1, pred_s2, pred_s3)
    boxes_t = jnp.pad(boxes, ((0, 0), (0, 32 - _NB), (0, 0)),
                      mode="edge").transpose(0, 2, 1)
    rows, cellidx = _sc_gather(boxes_t, pred_s1, pred_s2, pred_s3)
    t, c, o, n, cl = _assembly(parts1, parts2, parts3, boxes,
                               labels.astype(jnp.int32), rows, cellidx)
    return (t.reshape(()), c.reshape(()), o.reshape(()),
            n.reshape(()), cl.reshape(()))
